# Initial kernel scaffold; baseline (speedup 1.0000x reference)
import jax
import jax.numpy as jnp
from jax.experimental import pallas as pl


def kernel(x, t0, t1, t2):
    raise NotImplementedError("write your pallas kernel here")


# speedup vs baseline: 1.2476x; 1.2476x over previous
import functools

import jax
import jax.numpy as jnp
from jax import lax
from jax.experimental import pallas as pl
from jax.experimental.pallas import tpu as pltpu
from jax.experimental.pallas import tpu_sc as plsc

N = 100000
D = 128
C = 128
N_PAD = ((N + C - 1) // C) * C
NCHUNK = N_PAD // C
TAIL = N - (NCHUNK - 1) * C

_info = plsc.get_sparse_core_info()
NC, NS = _info.num_cores, _info.num_subcores
NW = NC * NS
STEPS = (NCHUNK + NW - 1) // NW


def _body(x0, x1, x2, t0, t1, t2, out, i0, i1, i2, r0, r1, r2, sem):
    wid = lax.axis_index("s") * NC + lax.axis_index("c")

    def step(k, carry):
        c = wid + k * NW

        @pl.when(c < NCHUNK)
        def _():
            base = c * C
            pltpu.sync_copy(x0.at[pl.ds(base, C)], i0)
            pltpu.sync_copy(x1.at[pl.ds(base, C)], i1)
            pltpu.sync_copy(x2.at[pl.ds(base, C)], i2)
            g0 = pltpu.async_copy(t0.at[i0], r0, sem)
            g1 = pltpu.async_copy(t1.at[i1], r1, sem)
            g2 = pltpu.async_copy(t2.at[i2], r2, sem)
            g0.wait()
            g1.wait()
            g2.wait()

            @pl.when(c < NCHUNK - 1)
            def _():
                pltpu.sync_copy(r0, out.at[pl.ds(base, C), pl.ds(0, D)])
                pltpu.sync_copy(r1, out.at[pl.ds(base, C), pl.ds(D, D)])
                pltpu.sync_copy(r2, out.at[pl.ds(base, C), pl.ds(2 * D, D)])

            @pl.when(c == NCHUNK - 1)
            def _():
                pltpu.sync_copy(r0.at[pl.ds(0, TAIL), :],
                                out.at[pl.ds(base, TAIL), pl.ds(0, D)])
                pltpu.sync_copy(r1.at[pl.ds(0, TAIL), :],
                                out.at[pl.ds(base, TAIL), pl.ds(D, D)])
                pltpu.sync_copy(r2.at[pl.ds(0, TAIL), :],
                                out.at[pl.ds(base, TAIL), pl.ds(2 * D, D)])

        return carry

    lax.fori_loop(0, STEPS, step, 0)


@functools.partial(jax.jit, static_argnames=())
def _run(x0, x1, x2, t0, t1, t2):
    mesh = plsc.VectorSubcoreMesh(core_axis_name="c", subcore_axis_name="s")
    f = pl.kernel(
        _body,
        out_type=jax.ShapeDtypeStruct((N, 3 * D), jnp.float32),
        mesh=mesh,
        scratch_types=[
            pltpu.VMEM((C,), jnp.int32),
            pltpu.VMEM((C,), jnp.int32),
            pltpu.VMEM((C,), jnp.int32),
            pltpu.VMEM((C, D), jnp.float32),
            pltpu.VMEM((C, D), jnp.float32),
            pltpu.VMEM((C, D), jnp.float32),
            pltpu.SemaphoreType.DMA,
        ],
    )
    return f(x0, x1, x2, t0, t1, t2)


def kernel(x, t0, t1, t2):
    xi = x.astype(jnp.int32)
    pad = N_PAD - N
    x0 = jnp.pad(xi[:, 0], (0, pad))
    x1 = jnp.pad(xi[:, 1], (0, pad))
    x2 = jnp.pad(xi[:, 2], (0, pad))
    return _run(x0, x1, x2, t0, t1, t2)


# speedup vs baseline: 1.2492x; 1.0013x over previous
import functools

import jax
import jax.numpy as jnp
from jax import lax
from jax.experimental import pallas as pl
from jax.experimental.pallas import tpu as pltpu
from jax.experimental.pallas import tpu_sc as plsc

N = 100000
D = 128
C = 128
N_PAD = ((N + C - 1) // C) * C
NCHUNK = N_PAD // C
TAIL = N - (NCHUNK - 1) * C

_info = plsc.get_sparse_core_info()
NC, NS = _info.num_cores, _info.num_subcores
NW = NC * NS
STEPS = (NCHUNK + NW - 1) // NW
NBUF = 2
OUTER = (STEPS + NBUF - 1) // NBUF


def _body(xt, t0, t1, t2, out, i0, i1, r00, r10, r20, r01, r11, r21,
          gs0, gs1, ws0, ws1):
    idxv = [i0, i1]
    rows = [[r00, r10, r20], [r01, r11, r21]]
    gsem = [gs0, gs1]
    wsem = [ws0, ws1]
    tabs = [t0, t1, t2]
    wid = lax.axis_index("s") * NC + lax.axis_index("c")

    def outer(i, carry):
        for b in range(NBUF):
            kk = i * NBUF + b
            c = wid + kk * NW
            prev_c = c - NBUF * NW

            @pl.when(jnp.logical_and(kk >= NBUF, prev_c < NCHUNK - 1))
            def _(b=b):
                for t in range(3):
                    pltpu.make_async_copy(
                        rows[b][t],
                        out.at[pl.ds(0, C), pl.ds(t * D, D)],
                        wsem[b],
                    ).wait()

            @pl.when(c < NCHUNK)
            def _(b=b, c=c):
                base = c * C
                pltpu.sync_copy(xt.at[:, pl.ds(base, C)], idxv[b])
                gs = [
                    pltpu.async_copy(tabs[t].at[idxv[b].at[t]],
                                     rows[b][t], gsem[b])
                    for t in range(3)
                ]
                for g in gs:
                    g.wait()

                @pl.when(c < NCHUNK - 1)
                def _(b=b):
                    for t in range(3):
                        pltpu.async_copy(
                            rows[b][t],
                            out.at[pl.ds(base, C), pl.ds(t * D, D)],
                            wsem[b],
                        )

                @pl.when(c == NCHUNK - 1)
                def _(b=b):
                    for t in range(3):
                        pltpu.sync_copy(
                            rows[b][t].at[pl.ds(0, TAIL), :],
                            out.at[pl.ds(base, TAIL), pl.ds(t * D, D)],
                        )

        return carry

    lax.fori_loop(0, OUTER, outer, 0)

    last_c = wid + (STEPS - 1) * NW

    @pl.when(last_c < NCHUNK - 1)
    def _():
        for t in range(3):
            pltpu.make_async_copy(
                rows[0][t],
                out.at[pl.ds(0, C), pl.ds(t * D, D)],
                wsem[0],
            ).wait()


@jax.jit
def _run(xt, t0, t1, t2):
    mesh = plsc.VectorSubcoreMesh(core_axis_name="c", subcore_axis_name="s")
    f = pl.kernel(
        _body,
        out_type=jax.ShapeDtypeStruct((N, 3 * D), jnp.float32),
        mesh=mesh,
        scratch_types=[
            pltpu.VMEM((3, C), jnp.int32),
            pltpu.VMEM((3, C), jnp.int32),
            pltpu.VMEM((C, D), jnp.float32),
            pltpu.VMEM((C, D), jnp.float32),
            pltpu.VMEM((C, D), jnp.float32),
            pltpu.VMEM((C, D), jnp.float32),
            pltpu.VMEM((C, D), jnp.float32),
            pltpu.VMEM((C, D), jnp.float32),
            pltpu.SemaphoreType.DMA,
            pltpu.SemaphoreType.DMA,
            pltpu.SemaphoreType.DMA,
            pltpu.SemaphoreType.DMA,
        ],
    )
    return f(xt, t0, t1, t2)


def kernel(x, t0, t1, t2):
    xt = jnp.pad(x.astype(jnp.int32).T, ((0, 0), (0, N_PAD - N)))
    return _run(xt, t0, t1, t2)
